# baseline (speedup 1.0000x reference)
import functools

import jax
import jax.numpy as jnp
from jax import lax
from jax.experimental import pallas as pl
from jax.experimental.pallas import tpu as pltpu
from jax.experimental.pallas import tpu_sc as plsc

_H = 16
_TOTAL_B = 64
_QL = 2048
_KL = 2048
_LINE = 4096
_QBLK = 128
_NSEM = 4

_THRESH = (19, 21, 24, 27, 31, 35, 40, 46, 52, 59, 67, 77, 87, 99, 113)

_LANES = 16
_NW = 32
_CHUNK = (_H * _LINE) // _NW


def _sc_body(emb_hbm, delta_hbm, lines_hbm, emb_v, delta_v, out_v):
    cid = lax.axis_index("c")
    sid = lax.axis_index("s")
    wid = sid * 2 + cid
    h = wid // 2
    base = (wid % 2) * _CHUNK
    pltpu.sync_copy(emb_hbm, emb_v)
    pltpu.sync_copy(delta_hbm, delta_v)
    delta = delta_v[...]
    lane = lax.iota(jnp.int32, _LANES)
    zero = jnp.full((_LANES,), 0, jnp.int32)
    sixteen = jnp.full((_LANES,), 16, jnp.int32)
    thirty_two = jnp.full((_LANES,), 32, jnp.int32)

    def step(t, carry):
        i = lane + (base + t * _LANES)
        rel = i - jnp.int32(_KL - 1) + delta
        r = jnp.abs(rel)
        large = sixteen
        for thr in _THRESH:
            large = jnp.where(r >= thr, large + 1, large)
        b = jnp.where(r < sixteen, r, large) + jnp.where(rel > zero, thirty_two, zero)
        out_v[pl.ds(t * _LANES, _LANES)] = plsc.load_gather(emb_v, [b * _H + h])
        return carry

    lax.fori_loop(0, _CHUNK // _LANES, step, 0)
    pltpu.sync_copy(out_v, lines_hbm.at[pl.ds(wid * _CHUNK, _CHUNK)])


def _sc_lines(emb_flat, delta_vec):
    mesh = plsc.VectorSubcoreMesh(core_axis_name="c", subcore_axis_name="s")
    fn = functools.partial(
        pl.kernel,
        out_type=jax.ShapeDtypeStruct((_H * _LINE,), jnp.float32),
        mesh=mesh,
        scratch_types=[
            pltpu.VMEM((_TOTAL_B * _H,), jnp.float32),
            pltpu.VMEM((_LANES,), jnp.int32),
            pltpu.VMEM((_CHUNK,), jnp.float32),
        ],
        compiler_params=pltpu.CompilerParams(needs_layout_passes=False),
    )(_sc_body)
    return fn(emb_flat, delta_vec).reshape(_H, _LINE)


def _tc_body(lines_ref, out_ref, m_ref, sems):
    h = pl.program_id(0)
    a = pl.program_id(1)
    i = h * (_QL // _QBLK) + a
    n = _H * (_QL // _QBLK)

    mh = m_ref.at[h % 2]

    @pl.when(a == 0)
    def _():
        mh[_QBLK - 1, :] = lines_ref[h, :]
        for j in range(7):
            w = 1 << j
            src = mh[_QBLK - w:_QBLK, :]
            mh[_QBLK - 2 * w:_QBLK - w, :] = pltpu.roll(src, _LINE - w, 1)

    u0 = pl.multiple_of((_QL - _QBLK) - _QBLK * a, _QBLK)
    src = mh.at[:, pl.ds(u0, _KL)]
    dst = out_ref.at[h, pl.ds(a * _QBLK, _QBLK), :]

    @pl.when(i >= _NSEM)
    def _():
        pltpu.make_async_copy(src, dst, sems.at[i % _NSEM]).wait()

    pltpu.make_async_copy(src, dst, sems.at[i % _NSEM]).start()

    @pl.when(i == n - 1)
    def _():
        for t in range(_NSEM):
            pltpu.make_async_copy(src, dst, sems.at[t]).wait()


def _tc_expand(lines):
    return pl.pallas_call(
        _tc_body,
        grid=(_H, _QL // _QBLK),
        in_specs=[pl.BlockSpec((_H, _LINE), lambda h, a: (0, 0))],
        out_specs=pl.BlockSpec(memory_space=pl.ANY),
        out_shape=jax.ShapeDtypeStruct((_H, _QL, _KL), jnp.float32),
        scratch_shapes=[
            pltpu.VMEM((2, _QBLK, _LINE), jnp.float32),
            pltpu.SemaphoreType.DMA((_NSEM,)),
        ],
    )(lines)


def kernel(q_len, k_len, emb):
    delta = jnp.asarray(k_len, jnp.int32) - jnp.asarray(q_len, jnp.int32)
    delta_vec = jnp.broadcast_to(delta, (_LANES,))
    lines = _sc_lines(emb.reshape(_TOTAL_B * _H), delta_vec)
    return _tc_expand(lines)[None]


# speedup vs baseline: 1.0230x; 1.0230x over previous
import functools

import jax
import jax.numpy as jnp
from jax import lax
from jax.experimental import pallas as pl
from jax.experimental.pallas import tpu as pltpu
from jax.experimental.pallas import tpu_sc as plsc

_H = 16
_TOTAL_B = 64
_QL = 2048
_KL = 2048
_LINE = 4096
_QBLK = 128
_NSEM = 4

_THRESH = (19, 21, 24, 27, 31, 35, 40, 46, 52, 59, 67, 77, 87, 99, 113)

_LANES = 16
_NW = 32
_IBLK = _LINE // _NW


def _sc_body(emb_hbm, delta_hbm, lines_hbm, emb_v, delta_v, out_v):
    cid = lax.axis_index("c")
    sid = lax.axis_index("s")
    wid = sid * 2 + cid
    i0 = wid * _IBLK
    pltpu.sync_copy(emb_hbm, emb_v)
    pltpu.sync_copy(delta_hbm, delta_v)
    delta = delta_v[...]
    lane = lax.iota(jnp.int32, _LANES)
    zero = jnp.full((_LANES,), 0, jnp.int32)
    sixteen = jnp.full((_LANES,), 16, jnp.int32)
    thirty_two = jnp.full((_LANES,), 32, jnp.int32)

    def step(t, carry):
        i = lane + (i0 + t * _LANES)
        rel = i - jnp.int32(_KL - 1) + delta
        r = jnp.abs(rel)
        large = sixteen
        for thr in _THRESH:
            large = jnp.where(r >= thr, large + 1, large)
        b = jnp.where(r < sixteen, r, large) + jnp.where(rel > zero, thirty_two, zero)
        bh = b * _H
        for h2 in range(_H):
            out_v[h2, pl.ds(t * _LANES, _LANES)] = plsc.load_gather(emb_v, [bh + h2])
        return carry

    lax.fori_loop(0, _IBLK // _LANES, step, 0)
    for h2 in range(_H):
        pltpu.sync_copy(out_v.at[h2], lines_hbm.at[pl.ds(h2 * _LINE + i0, _IBLK)])


def _sc_lines(emb_flat, delta_vec):
    mesh = plsc.VectorSubcoreMesh(core_axis_name="c", subcore_axis_name="s")
    fn = functools.partial(
        pl.kernel,
        out_type=jax.ShapeDtypeStruct((_H * _LINE,), jnp.float32),
        mesh=mesh,
        scratch_types=[
            pltpu.VMEM((_TOTAL_B * _H,), jnp.float32),
            pltpu.VMEM((_LANES,), jnp.int32),
            pltpu.VMEM((_H, _IBLK), jnp.float32),
        ],
        compiler_params=pltpu.CompilerParams(needs_layout_passes=False),
    )(_sc_body)
    return fn(emb_flat, delta_vec).reshape(_H, _LINE)


def _tc_body(lines_ref, out_ref, m_ref, sems):
    h = pl.program_id(0)
    a = pl.program_id(1)
    i = h * (_QL // _QBLK) + a
    n = _H * (_QL // _QBLK)

    mh = m_ref.at[h % 2]

    @pl.when(a == 0)
    def _():
        mh[_QBLK - 1, :] = lines_ref[h, :]
        for j in range(7):
            w = 1 << j
            src = mh[_QBLK - w:_QBLK, :]
            mh[_QBLK - 2 * w:_QBLK - w, :] = pltpu.roll(src, _LINE - w, 1)

    u0 = pl.multiple_of((_QL - _QBLK) - _QBLK * a, _QBLK)
    src = mh.at[:, pl.ds(u0, _KL)]
    dst = out_ref.at[h, pl.ds(a * _QBLK, _QBLK), :]

    @pl.when(i >= _NSEM)
    def _():
        pltpu.make_async_copy(src, dst, sems.at[i % _NSEM]).wait()

    pltpu.make_async_copy(src, dst, sems.at[i % _NSEM]).start()

    @pl.when(i == n - 1)
    def _():
        for t in range(_NSEM):
            pltpu.make_async_copy(src, dst, sems.at[t]).wait()


def _tc_expand(lines):
    return pl.pallas_call(
        _tc_body,
        grid=(_H, _QL // _QBLK),
        in_specs=[pl.BlockSpec((_H, _LINE), lambda h, a: (0, 0))],
        out_specs=pl.BlockSpec(memory_space=pl.ANY),
        out_shape=jax.ShapeDtypeStruct((_H, _QL, _KL), jnp.float32),
        scratch_shapes=[
            pltpu.VMEM((2, _QBLK, _LINE), jnp.float32),
            pltpu.SemaphoreType.DMA((_NSEM,)),
        ],
    )(lines)


def kernel(q_len, k_len, emb):
    delta = jnp.asarray(k_len, jnp.int32) - jnp.asarray(q_len, jnp.int32)
    delta_vec = jnp.broadcast_to(delta, (_LANES,))
    lines = _sc_lines(emb.reshape(_TOTAL_B * _H), delta_vec)
    return _tc_expand(lines)[None]


# speedup vs baseline: 1.0591x; 1.0353x over previous
import functools

import jax
import jax.numpy as jnp
from jax import lax
from jax.experimental import pallas as pl
from jax.experimental.pallas import tpu as pltpu
from jax.experimental.pallas import tpu_sc as plsc

_H = 16
_TOTAL_B = 64
_QL = 2048
_KL = 2048
_LINE = 4096
_QBLK = 128
_NSEM = 8

_THRESH = (19, 21, 24, 27, 31, 35, 40, 46, 52, 59, 67, 77, 87, 99, 113)

_LANES = 16
_NW = 32
_IBLK = _LINE // _NW


def _sc_body(emb_hbm, delta_hbm, lines_hbm, emb_v, delta_v, out_v):
    cid = lax.axis_index("c")
    sid = lax.axis_index("s")
    wid = sid * 2 + cid
    i0 = wid * _IBLK
    pltpu.sync_copy(emb_hbm, emb_v)
    pltpu.sync_copy(delta_hbm, delta_v)
    delta = delta_v[...]
    lane = lax.iota(jnp.int32, _LANES)
    zero = jnp.full((_LANES,), 0, jnp.int32)
    sixteen = jnp.full((_LANES,), 16, jnp.int32)
    thirty_two = jnp.full((_LANES,), 32, jnp.int32)

    def step(t, carry):
        i = lane + (i0 + t * _LANES)
        rel = i - jnp.int32(_KL - 1) + delta
        r = jnp.abs(rel)
        large = sixteen
        for thr in _THRESH:
            large = jnp.where(r >= thr, large + 1, large)
        b = jnp.where(r < sixteen, r, large) + jnp.where(rel > zero, thirty_two, zero)
        bh = b * _H
        for h2 in range(_H):
            out_v[h2, pl.ds(t * _LANES, _LANES)] = plsc.load_gather(emb_v, [bh + h2])
        return carry

    lax.fori_loop(0, _IBLK // _LANES, step, 0)
    for h2 in range(_H):
        pltpu.sync_copy(out_v.at[h2], lines_hbm.at[pl.ds(h2 * _LINE + i0, _IBLK)])


def _sc_lines(emb_flat, delta_vec):
    mesh = plsc.VectorSubcoreMesh(core_axis_name="c", subcore_axis_name="s")
    fn = functools.partial(
        pl.kernel,
        out_type=jax.ShapeDtypeStruct((_H * _LINE,), jnp.float32),
        mesh=mesh,
        scratch_types=[
            pltpu.VMEM((_TOTAL_B * _H,), jnp.float32),
            pltpu.VMEM((_LANES,), jnp.int32),
            pltpu.VMEM((_H, _IBLK), jnp.float32),
        ],
        compiler_params=pltpu.CompilerParams(needs_layout_passes=False),
    )(_sc_body)
    return fn(emb_flat, delta_vec).reshape(_H, _LINE)


def _tc_body(lines_ref, out_ref, m_ref, sems):
    h = pl.program_id(0)
    a = pl.program_id(1)
    i = h * (_QL // _QBLK) + a
    n = _H * (_QL // _QBLK)

    mh = m_ref.at[h % 2]

    @pl.when(a == 0)
    def _():
        mh[_QBLK - 1, :] = lines_ref[h, :]
        for j in range(7):
            w = 1 << j
            src = mh[_QBLK - w:_QBLK, :]
            mh[_QBLK - 2 * w:_QBLK - w, :] = pltpu.roll(src, _LINE - w, 1)

    u0 = pl.multiple_of((_QL - _QBLK) - _QBLK * a, _QBLK)
    src = mh.at[:, pl.ds(u0, _KL)]
    dst = out_ref.at[h, pl.ds(a * _QBLK, _QBLK), :]

    @pl.when(i >= _NSEM)
    def _():
        pltpu.make_async_copy(src, dst, sems.at[i % _NSEM]).wait()

    pltpu.make_async_copy(src, dst, sems.at[i % _NSEM]).start()

    @pl.when(i == n - 1)
    def _():
        for t in range(_NSEM):
            pltpu.make_async_copy(src, dst, sems.at[t]).wait()


def _tc_expand(lines):
    return pl.pallas_call(
        _tc_body,
        grid=(_H, _QL // _QBLK),
        in_specs=[pl.BlockSpec((_H, _LINE), lambda h, a: (0, 0))],
        out_specs=pl.BlockSpec(memory_space=pl.ANY),
        out_shape=jax.ShapeDtypeStruct((_H, _QL, _KL), jnp.float32),
        scratch_shapes=[
            pltpu.VMEM((2, _QBLK, _LINE), jnp.float32),
            pltpu.SemaphoreType.DMA((_NSEM,)),
        ],
    )(lines)


def kernel(q_len, k_len, emb):
    delta = jnp.asarray(k_len, jnp.int32) - jnp.asarray(q_len, jnp.int32)
    delta_vec = jnp.broadcast_to(delta, (_LANES,))
    lines = _sc_lines(emb.reshape(_TOTAL_B * _H), delta_vec)
    return _tc_expand(lines)[None]


# speedup vs baseline: 1.0604x; 1.0013x over previous
import functools

import jax
import jax.numpy as jnp
from jax import lax
from jax.experimental import pallas as pl
from jax.experimental.pallas import tpu as pltpu
from jax.experimental.pallas import tpu_sc as plsc

_H = 16
_TOTAL_B = 64
_QL = 2048
_KL = 2048
_LINE = 4096
_QBLK = 128
_NSEM = 16

_THRESH = (19, 21, 24, 27, 31, 35, 40, 46, 52, 59, 67, 77, 87, 99, 113)

_LANES = 16
_NW = 32
_IBLK = _LINE // _NW


def _sc_body(emb_hbm, delta_hbm, lines_hbm, emb_v, delta_v, out_v):
    cid = lax.axis_index("c")
    sid = lax.axis_index("s")
    wid = sid * 2 + cid
    i0 = wid * _IBLK
    pltpu.sync_copy(emb_hbm, emb_v)
    pltpu.sync_copy(delta_hbm, delta_v)
    delta = delta_v[...]
    lane = lax.iota(jnp.int32, _LANES)
    zero = jnp.full((_LANES,), 0, jnp.int32)
    sixteen = jnp.full((_LANES,), 16, jnp.int32)
    thirty_two = jnp.full((_LANES,), 32, jnp.int32)

    def step(t, carry):
        i = lane + (i0 + t * _LANES)
        rel = i - jnp.int32(_KL - 1) + delta
        r = jnp.abs(rel)
        large = sixteen
        for thr in _THRESH:
            large = jnp.where(r >= thr, large + 1, large)
        b = jnp.where(r < sixteen, r, large) + jnp.where(rel > zero, thirty_two, zero)
        bh = b * _H
        for h2 in range(_H):
            out_v[h2, pl.ds(t * _LANES, _LANES)] = plsc.load_gather(emb_v, [bh + h2])
        return carry

    lax.fori_loop(0, _IBLK // _LANES, step, 0)
    for h2 in range(_H):
        pltpu.sync_copy(out_v.at[h2], lines_hbm.at[pl.ds(h2 * _LINE + i0, _IBLK)])


def _sc_lines(emb_flat, delta_vec):
    mesh = plsc.VectorSubcoreMesh(core_axis_name="c", subcore_axis_name="s")
    fn = functools.partial(
        pl.kernel,
        out_type=jax.ShapeDtypeStruct((_H * _LINE,), jnp.float32),
        mesh=mesh,
        scratch_types=[
            pltpu.VMEM((_TOTAL_B * _H,), jnp.float32),
            pltpu.VMEM((_LANES,), jnp.int32),
            pltpu.VMEM((_H, _IBLK), jnp.float32),
        ],
        compiler_params=pltpu.CompilerParams(needs_layout_passes=False),
    )(_sc_body)
    return fn(emb_flat, delta_vec).reshape(_H, _LINE)


def _tc_body(lines_ref, out_ref, m_ref, sems):
    h = pl.program_id(0)
    a = pl.program_id(1)
    i = h * (_QL // _QBLK) + a
    n = _H * (_QL // _QBLK)

    mh = m_ref.at[h % 2]

    @pl.when(a == 0)
    def _():
        mh[_QBLK - 1, :] = lines_ref[h, :]
        for j in range(7):
            w = 1 << j
            src = mh[_QBLK - w:_QBLK, :]
            mh[_QBLK - 2 * w:_QBLK - w, :] = pltpu.roll(src, _LINE - w, 1)

    u0 = pl.multiple_of((_QL - _QBLK) - _QBLK * a, _QBLK)
    src = mh.at[:, pl.ds(u0, _KL)]
    dst = out_ref.at[h, pl.ds(a * _QBLK, _QBLK), :]

    @pl.when(i >= _NSEM)
    def _():
        pltpu.make_async_copy(src, dst, sems.at[i % _NSEM]).wait()

    pltpu.make_async_copy(src, dst, sems.at[i % _NSEM]).start()

    @pl.when(i == n - 1)
    def _():
        for t in range(_NSEM):
            pltpu.make_async_copy(src, dst, sems.at[t]).wait()


def _tc_expand(lines):
    return pl.pallas_call(
        _tc_body,
        grid=(_H, _QL // _QBLK),
        in_specs=[pl.BlockSpec((_H, _LINE), lambda h, a: (0, 0))],
        out_specs=pl.BlockSpec(memory_space=pl.ANY),
        out_shape=jax.ShapeDtypeStruct((_H, _QL, _KL), jnp.float32),
        scratch_shapes=[
            pltpu.VMEM((2, _QBLK, _LINE), jnp.float32),
            pltpu.SemaphoreType.DMA((_NSEM,)),
        ],
    )(lines)


def kernel(q_len, k_len, emb):
    delta = jnp.asarray(k_len, jnp.int32) - jnp.asarray(q_len, jnp.int32)
    delta_vec = jnp.broadcast_to(delta, (_LANES,))
    lines = _sc_lines(emb.reshape(_TOTAL_B * _H), delta_vec)
    return _tc_expand(lines)[None]


# speedup vs baseline: 1.0624x; 1.0019x over previous
import functools

import jax
import jax.numpy as jnp
from jax import lax
from jax.experimental import pallas as pl
from jax.experimental.pallas import tpu as pltpu
from jax.experimental.pallas import tpu_sc as plsc

_H = 16
_TOTAL_B = 64
_QL = 2048
_KL = 2048
_LINE = 4096
_QBLK = 256
_NSEM = 8

_THRESH = (19, 21, 24, 27, 31, 35, 40, 46, 52, 59, 67, 77, 87, 99, 113)

_LANES = 16
_NW = 32
_IBLK = _LINE // _NW


def _sc_body(emb_hbm, delta_hbm, lines_hbm, emb_v, delta_v, out_v):
    cid = lax.axis_index("c")
    sid = lax.axis_index("s")
    wid = sid * 2 + cid
    i0 = wid * _IBLK
    pltpu.sync_copy(emb_hbm, emb_v)
    pltpu.sync_copy(delta_hbm, delta_v)
    delta = delta_v[...]
    lane = lax.iota(jnp.int32, _LANES)
    zero = jnp.full((_LANES,), 0, jnp.int32)
    sixteen = jnp.full((_LANES,), 16, jnp.int32)
    thirty_two = jnp.full((_LANES,), 32, jnp.int32)

    def step(t, carry):
        i = lane + (i0 + t * _LANES)
        rel = i - jnp.int32(_KL - 1) + delta
        r = jnp.abs(rel)
        large = sixteen
        for thr in _THRESH:
            large = jnp.where(r >= thr, large + 1, large)
        b = jnp.where(r < sixteen, r, large) + jnp.where(rel > zero, thirty_two, zero)
        bh = b * _H
        for h2 in range(_H):
            out_v[h2, pl.ds(t * _LANES, _LANES)] = plsc.load_gather(emb_v, [bh + h2])
        return carry

    lax.fori_loop(0, _IBLK // _LANES, step, 0)
    for h2 in range(_H):
        pltpu.sync_copy(out_v.at[h2], lines_hbm.at[pl.ds(h2 * _LINE + i0, _IBLK)])


def _sc_lines(emb_flat, delta_vec):
    mesh = plsc.VectorSubcoreMesh(core_axis_name="c", subcore_axis_name="s")
    fn = functools.partial(
        pl.kernel,
        out_type=jax.ShapeDtypeStruct((_H * _LINE,), jnp.float32),
        mesh=mesh,
        scratch_types=[
            pltpu.VMEM((_TOTAL_B * _H,), jnp.float32),
            pltpu.VMEM((_LANES,), jnp.int32),
            pltpu.VMEM((_H, _IBLK), jnp.float32),
        ],
        compiler_params=pltpu.CompilerParams(needs_layout_passes=False),
    )(_sc_body)
    return fn(emb_flat, delta_vec).reshape(_H, _LINE)


def _tc_body(lines_ref, out_ref, m_ref, sems):
    h = pl.program_id(0)
    a = pl.program_id(1)
    i = h * (_QL // _QBLK) + a
    n = _H * (_QL // _QBLK)

    mh = m_ref.at[h % 2]

    @pl.when(a == 0)
    def _():
        mh[_QBLK - 1, :] = lines_ref[h, :]
        for j in range(8):
            w = 1 << j
            src = mh[_QBLK - w:_QBLK, :]
            mh[_QBLK - 2 * w:_QBLK - w, :] = pltpu.roll(src, _LINE - w, 1)

    u0 = pl.multiple_of((_QL - _QBLK) - _QBLK * a, _QBLK)
    src = mh.at[:, pl.ds(u0, _KL)]
    dst = out_ref.at[h, pl.ds(a * _QBLK, _QBLK), :]

    @pl.when(i >= _NSEM)
    def _():
        pltpu.make_async_copy(src, dst, sems.at[i % _NSEM]).wait()

    pltpu.make_async_copy(src, dst, sems.at[i % _NSEM]).start()

    @pl.when(i == n - 1)
    def _():
        for t in range(_NSEM):
            pltpu.make_async_copy(src, dst, sems.at[t]).wait()


def _tc_expand(lines):
    return pl.pallas_call(
        _tc_body,
        grid=(_H, _QL // _QBLK),
        in_specs=[pl.BlockSpec((_H, _LINE), lambda h, a: (0, 0))],
        out_specs=pl.BlockSpec(memory_space=pl.ANY),
        out_shape=jax.ShapeDtypeStruct((_H, _QL, _KL), jnp.float32),
        scratch_shapes=[
            pltpu.VMEM((2, _QBLK, _LINE), jnp.float32),
            pltpu.SemaphoreType.DMA((_NSEM,)),
        ],
    )(lines)


def kernel(q_len, k_len, emb):
    delta = jnp.asarray(k_len, jnp.int32) - jnp.asarray(q_len, jnp.int32)
    delta_vec = jnp.broadcast_to(delta, (_LANES,))
    lines = _sc_lines(emb.reshape(_TOTAL_B * _H), delta_vec)
    return _tc_expand(lines)[None]
